# baseline (speedup 1.0000x reference)
import jax
import jax.numpy as jnp
from jax import lax
from jax.experimental import pallas as pl
from jax.experimental.pallas import tpu as pltpu
from jax.experimental.pallas import tpu_sc as plsc

F32 = jnp.float32
I32 = jnp.int32

W = 2048
LB = 128
LISTN = W + LB


def _sc_conv(tbl, src, sel, dst, itp, clus, N, co):
    E = src.shape[0]
    co_t = min(co, 128)
    nsl = co // co_t
    Nc = min(N, 65536 // co_t)
    chunks = N // Nc
    combos = nsl * chunks
    P = max(1, combos // 32)
    G = max(1, 32 // combos)
    Eg = E // G
    nwin, rem = Eg // W, Eg % W
    has_clus = clus is not None
    packed = co < 128
    nb = co_t // 16

    def body(*refs):
        if has_clus:
            (tbl_r, src_r, sel_r, dst_r, itp_r, clus_r, out_r,
             clusv, srcw, selw, dstw, itpw, idxl, dl, itl, ofl,
             gbuf, acc, sem) = refs
        else:
            (tbl_r, src_r, sel_r, dst_r, itp_r, out_r,
             srcw, selw, dstw, itpw, idxl, dl, itl, ofl,
             gbuf, acc, sem) = refs
        IOTA = lax.iota(I32, 16)
        cid = lax.axis_index("c")
        sid = lax.axis_index("s")
        wid = sid * 2 + cid
        if has_clus:
            pltpu.sync_copy(clus_r, clusv)

        gbuff = gbuf.reshape(LB * 128)

        def proc_blocks(nblk):
            def _blk(k, _):
                pltpu.async_copy(tbl_r.at[idxl.at[pl.ds(k * LB, LB)]],
                                 gbuf, sem).wait()

                def _edge(i, _):
                    for u in range(8):
                        e = i * 8 + u
                        ev = jnp.full((16,), k * LB + e, I32)
                        itb = plsc.load_gather(itl, [ev])
                        dlb = plsc.load_gather(dl, [ev])
                        if packed:
                            ofb = plsc.load_gather(ofl, [ev])
                            ev0 = jnp.full((16,), e, I32)
                            for c in range(nb):
                                cv = c * 16 + IOTA
                                val = plsc.load_gather(gbuf, [ev0, ofb + cv])
                                plsc.addupdate_scatter(
                                    acc, [dlb + cv], val * itb)
                        else:
                            for c in range(nb):
                                cv = c * 16 + IOTA
                                val = gbuf[e, pl.ds(c * 16, 16)]
                                plsc.addupdate_scatter(
                                    acc, [dlb + cv], val * itb)
                    return 0
                lax.fori_loop(0, LB // 8, _edge, 0)
                return 0
            lax.fori_loop(0, nblk, _blk, 0)

        def scan_window(wb, ww, j, lo):
            pltpu.sync_copy(src_r.at[pl.ds(wb, ww)], srcw.at[pl.ds(0, ww)])
            pltpu.sync_copy(sel_r.at[pl.ds(wb, ww)], selw.at[pl.ds(0, ww)])
            pltpu.sync_copy(dst_r.at[pl.ds(wb, ww)], dstw.at[pl.ds(0, ww)])
            pltpu.sync_copy(itp_r.at[pl.ds(wb, ww)], itpw.at[pl.ds(0, ww)])
            joff = j * co_t

            if chunks == 1:
                def _tr(k, _):
                    sl = pl.ds(k * 16, 16)
                    s16 = srcw[sl]
                    if has_clus:
                        s16 = plsc.load_gather(clusv, [s16])
                    pos = (s16 * 9 + selw[sl]) * co + joff
                    idxl[sl] = pos >> 7
                    if packed:
                        ofl[sl] = pos & 127
                    dl[sl] = dstw[sl] * co_t
                    itl[sl] = itpw[sl]
                    return 0
                lax.fori_loop(0, ww // 16, _tr, 0)
                proc_blocks(ww // LB)
            else:
                def _sc(k, cnt):
                    sl = pl.ds(k * 16, 16)
                    s16 = srcw[sl]
                    if has_clus:
                        s16 = plsc.load_gather(clusv, [s16])
                    pos = (s16 * 9 + selw[sl]) * co + joff
                    d16 = dstw[sl]
                    inb = (d16 >= lo) & (d16 < lo + Nc)
                    m = inb.astype(I32)
                    pp = cnt + plsc.cumsum(m) - m
                    plsc.store_scatter(idxl, [pp], pos >> 7, mask=inb)
                    if packed:
                        plsc.store_scatter(ofl, [pp], pos & 127, mask=inb)
                    plsc.store_scatter(dl, [pp], (d16 - lo) * co_t, mask=inb)
                    plsc.store_scatter(itl, [pp], itpw[sl], mask=inb)
                    return cnt + jnp.sum(m)
                cnt = lax.fori_loop(0, ww // 16, _sc, 0)
                for u in range(LB // 16):
                    pv = cnt + u * 16 + IOTA
                    plsc.store_scatter(idxl, [pv], IOTA + u * 16)
                    if packed:
                        plsc.store_scatter(ofl, [pv], jnp.zeros((16,), I32))
                    plsc.store_scatter(dl, [pv], jnp.full((16,), Nc * co_t, I32))
                    plsc.store_scatter(itl, [pv], jnp.zeros((16,), F32))
                proc_blocks((cnt + LB - 1) // LB)

        for p in range(P):
            combo = p * 32 + wid
            j = combo % nsl
            h = (combo // nsl) % chunks
            g = combo // combos
            lo = h * Nc

            def _z(t, _):
                for u in range(8):
                    acc[pl.ds((t * 8 + u) * 16, 16)] = jnp.zeros((16,), F32)
                return 0
            lax.fori_loop(0, 512, _z, 0)

            gbase = g * Eg

            def _win(wi, _):
                scan_window(gbase + wi * W, W, j, lo)
                return 0
            lax.fori_loop(0, nwin, _win, 0)
            if rem:
                scan_window(gbase + nwin * W, rem, j, lo)

            row = (g * chunks + h) * nsl + j
            pltpu.sync_copy(acc.at[pl.ds(0, Nc * co_t)], out_r.at[row])

    scratch = []
    if has_clus:
        scratch.append(pltpu.VMEM((clus.shape[0],), I32))
    scratch += [
        pltpu.VMEM((W,), I32),
        pltpu.VMEM((W,), I32),
        pltpu.VMEM((W,), I32),
        pltpu.VMEM((W,), F32),
        pltpu.VMEM((LISTN,), I32),
        pltpu.VMEM((LISTN,), I32),
        pltpu.VMEM((LISTN,), F32),
        pltpu.VMEM((LISTN,), I32),
        pltpu.VMEM((LB, 128), F32),
        pltpu.VMEM((65536 + co_t,), F32),
        pltpu.SemaphoreType.DMA,
    ]
    mesh = plsc.VectorSubcoreMesh(core_axis_name="c", subcore_axis_name="s")
    fn = pl.kernel(
        body,
        out_type=jax.ShapeDtypeStruct((32 * P, Nc * co_t), F32),
        mesh=mesh,
        scratch_types=scratch,
        compiler_params=pltpu.CompilerParams(needs_layout_passes=False),
        name=f"sc_conv_N{N}_co{co}",
    )
    args = (tbl, src, sel, dst, itp) + ((clus,) if has_clus else ())
    return fn(*args)


def _assemble(p, N, co):
    co_t = min(co, 128)
    nsl = co // co_t
    Nc = min(N, 65536 // co_t)
    chunks = N // Nc
    G = p.shape[0] // (chunks * nsl)
    return (p.reshape(G, chunks, nsl, Nc, co_t)
            .transpose(0, 1, 3, 2, 4)
            .reshape(G, N, co))


def _sc_gather_i32(table, idx):
    E = idx.shape[0]
    per = E // 32
    nw = per // W

    def body(tbl_r, idx_r, out_r, idxw, valw, sem):
        cid = lax.axis_index("c")
        sid = lax.axis_index("s")
        base = (sid * 2 + cid) * per

        def _w(k, _):
            b = base + k * W
            pltpu.sync_copy(idx_r.at[pl.ds(b, W)], idxw)
            pltpu.async_copy(tbl_r.at[idxw], valw, sem).wait()
            pltpu.sync_copy(valw, out_r.at[pl.ds(b, W)])
            return 0
        lax.fori_loop(0, nw, _w, 0)

    mesh = plsc.VectorSubcoreMesh(core_axis_name="c", subcore_axis_name="s")
    fn = pl.kernel(
        body,
        out_type=jax.ShapeDtypeStruct((E,), I32),
        mesh=mesh,
        scratch_types=[
            pltpu.VMEM((W,), I32),
            pltpu.VMEM((W,), I32),
            pltpu.SemaphoreType.DMA,
        ],
        compiler_params=pltpu.CompilerParams(needs_layout_passes=False),
        name="sc_gather_i32",
    )
    return fn(table, idx)


def _mm_plain(x, w, bm):
    M, Kd = x.shape
    C = w.shape[1]

    def body(x_r, w_r, o_r):
        o_r[...] = jnp.dot(x_r[...], w_r[...], preferred_element_type=F32)

    return pl.pallas_call(
        body,
        grid=(M // bm,),
        in_specs=[pl.BlockSpec((bm, Kd), lambda i: (i, 0)),
                  pl.BlockSpec((Kd, C), lambda i: (0, 0))],
        out_specs=pl.BlockSpec((bm, C), lambda i: (i, 0)),
        out_shape=jax.ShapeDtypeStruct((M, C), F32),
    )(x, w)


def _mm_sum(p, b, w, bm):
    G, M, Kd = p.shape
    C = w.shape[1]
    b2 = b.reshape(1, Kd)

    def body(p_r, b_r, w_r, o_r):
        x = p_r[0]
        for g in range(1, G):
            x = x + p_r[g]
        x = jnp.maximum(x + b_r[0], 0.0)
        o_r[...] = jnp.dot(x, w_r[...], preferred_element_type=F32)

    return pl.pallas_call(
        body,
        grid=(M // bm,),
        in_specs=[pl.BlockSpec((G, bm, Kd), lambda i: (0, i, 0)),
                  pl.BlockSpec((1, Kd), lambda i: (0, 0)),
                  pl.BlockSpec((Kd, C), lambda i: (0, 0))],
        out_specs=pl.BlockSpec((bm, C), lambda i: (i, 0)),
        out_shape=jax.ShapeDtypeStruct((M, C), F32),
    )(p, b2, w)


def _final_sum(p, b, bm):
    G, M, C = p.shape
    b2 = b.reshape(1, C)

    def body(p_r, b_r, o_r):
        x = p_r[0]
        for g in range(1, G):
            x = x + p_r[g]
        o_r[...] = x + b_r[0]

    return pl.pallas_call(
        body,
        grid=(M // bm,),
        in_specs=[pl.BlockSpec((G, bm, C), lambda i: (0, i, 0)),
                  pl.BlockSpec((1, C), lambda i: (0, 0))],
        out_specs=pl.BlockSpec((bm, C), lambda i: (i, 0)),
        out_shape=jax.ShapeDtypeStruct((M, C), F32),
    )(p, b2)


def _wr(Wt, co=None):
    if co is not None and Wt.shape[2] != co:
        Wt = jnp.pad(Wt, ((0, 0), (0, 0), (0, co - Wt.shape[2])))
    ci = Wt.shape[1]
    return Wt.transpose(1, 0, 2).reshape(ci, 9 * Wt.shape[2])


def kernel(x, edge_index_3, selections_3, interps_3, edge_index_2, selections_2, interps_2, edge_index_1, selections_1, interps_1, edge_index_0, selections_0, interps_0, clusters_2, clusters_1, clusters_0, W11, b11, W12, b12, W13, b13, W14, b14, W15, b15, W16, b16, W17, b17, W18, b18, W19, b19):
    s3, d3 = edge_index_3[0], edge_index_3[1]
    s2, d2 = edge_index_2[0], edge_index_2[1]
    s1, d1 = edge_index_1[0], edge_index_1[1]
    s0, d0 = edge_index_0[0], edge_index_0[1]

    cs0 = _sc_gather_i32(clusters_0, s0)

    def tv(y):
        return y.reshape(-1, 128)

    y = _mm_plain(x, _wr(W11), 1024)
    p = _sc_conv(tv(y), s3, selections_3, d3, interps_3, None, 1024, 256)
    p = _assemble(p, 1024, 256)
    y = _mm_sum(p, b11, _wr(W12), 1024)
    p = _sc_conv(tv(y), s2, selections_2, d2, interps_2, clusters_2, 4096, 256)
    p = _assemble(p, 4096, 256)
    y = _mm_sum(p, b12, _wr(W13), 1024)
    p = _sc_conv(tv(y), s2, selections_2, d2, interps_2, None, 4096, 256)
    p = _assemble(p, 4096, 256)
    y = _mm_sum(p, b13, _wr(W14), 1024)
    p = _sc_conv(tv(y), s2, selections_2, d2, interps_2, None, 4096, 256)
    p = _assemble(p, 4096, 256)
    y = _mm_sum(p, b14, _wr(W15), 1024)
    p = _sc_conv(tv(y), s2, selections_2, d2, interps_2, None, 4096, 128)
    p = _assemble(p, 4096, 128)
    y = _mm_sum(p, b15, _wr(W16), 1024)
    p = _sc_conv(tv(y), s1, selections_1, d1, interps_1, clusters_1, 16384, 128)
    p = _assemble(p, 16384, 128)
    y = _mm_sum(p, b16, _wr(W17), 2048)
    p = _sc_conv(tv(y), s1, selections_1, d1, interps_1, None, 16384, 64)
    p = _assemble(p, 16384, 64)
    y = _mm_sum(p, b17, _wr(W18), 2048)
    p = _sc_conv(tv(y), cs0, selections_0, d0, interps_0, None, 65536, 64)
    p = _assemble(p, 65536, 64)
    y = _mm_sum(p, b18, _wr(W19, 16), 4096)
    p = _sc_conv(tv(y), s0, selections_0, d0, interps_0, None, 65536, 16)
    p = _assemble(p, 65536, 16)
    out = _final_sum(p, jnp.pad(b19, (0, 13)), 4096)
    return out[:, :3]


# speedup vs baseline: 1.0005x; 1.0005x over previous
import jax
import jax.numpy as jnp
from jax import lax
from jax.experimental import pallas as pl
from jax.experimental.pallas import tpu as pltpu
from jax.experimental.pallas import tpu_sc as plsc

F32 = jnp.float32
I32 = jnp.int32

W = 2048
LB = 128
LISTN = W + LB


def _sc_conv(tbl, src, sel, dst, itp, clus, N, co):
    E = src.shape[0]
    co_t = min(co, 128)
    nsl = co // co_t
    Nc = min(N, 65536 // co_t)
    chunks = N // Nc
    combos = nsl * chunks
    P = max(1, combos // 32)
    G = max(1, 32 // combos)
    Eg = E // G
    nwin, rem = Eg // W, Eg % W
    has_clus = clus is not None
    packed = co < 128
    nb = co_t // 16

    def body(*refs):
        if has_clus:
            (tbl_r, src_r, sel_r, dst_r, itp_r, clus_r, out_r,
             clusv, srcw, selw, dstw, itpw, idxl, dl, itl, ofl,
             gbuf, acc, sem) = refs
        else:
            (tbl_r, src_r, sel_r, dst_r, itp_r, out_r,
             srcw, selw, dstw, itpw, idxl, dl, itl, ofl,
             gbuf, acc, sem) = refs
        IOTA = lax.iota(I32, 16)
        cid = lax.axis_index("c")
        sid = lax.axis_index("s")
        wid = sid * 2 + cid
        if has_clus:
            pltpu.sync_copy(clus_r, clusv)

        def proc_blocks(nblk):
            def _blk(k, _):
                pltpu.async_copy(tbl_r.at[idxl.at[pl.ds(k * LB, LB)]],
                                 gbuf, sem).wait()

                def _edge(i, _):
                    for u in range(8):
                        e = i * 8 + u
                        ev = jnp.full((16,), k * LB + e, I32)
                        itb = plsc.load_gather(itl, [ev])
                        dlb = plsc.load_gather(dl, [ev])
                        if packed:
                            ofb = plsc.load_gather(ofl, [ev])
                            ev0 = jnp.full((16,), e, I32)
                            for c in range(nb):
                                cv = c * 16 + IOTA
                                val = plsc.load_gather(gbuf, [ev0, ofb + cv])
                                plsc.addupdate_scatter(
                                    acc, [dlb + cv], val * itb)
                        else:
                            for c in range(nb):
                                cv = c * 16 + IOTA
                                val = gbuf[e, pl.ds(c * 16, 16)]
                                plsc.addupdate_scatter(
                                    acc, [dlb + cv], val * itb)
                    return 0
                lax.fori_loop(0, LB // 8, _edge, 0)
                return 0
            lax.fori_loop(0, nblk, _blk, 0)

        def scan_window(wb, ww, j, lo):
            pltpu.sync_copy(src_r.at[pl.ds(wb, ww)], srcw.at[pl.ds(0, ww)])
            pltpu.sync_copy(sel_r.at[pl.ds(wb, ww)], selw.at[pl.ds(0, ww)])
            pltpu.sync_copy(dst_r.at[pl.ds(wb, ww)], dstw.at[pl.ds(0, ww)])
            pltpu.sync_copy(itp_r.at[pl.ds(wb, ww)], itpw.at[pl.ds(0, ww)])
            joff = j * co_t

            if chunks == 1:
                def _tr(k, _):
                    sl = pl.ds(k * 16, 16)
                    s16 = srcw[sl]
                    if has_clus:
                        s16 = plsc.load_gather(clusv, [s16])
                    pos = (s16 * 9 + selw[sl]) * co + joff
                    idxl[sl] = pos >> 7
                    if packed:
                        ofl[sl] = pos & 127
                    dl[sl] = dstw[sl] * co_t
                    itl[sl] = itpw[sl]
                    return 0
                lax.fori_loop(0, ww // 16, _tr, 0)
                proc_blocks(ww // LB)
            else:
                def _sc(k, cnt):
                    sl = pl.ds(k * 16, 16)
                    s16 = srcw[sl]
                    if has_clus:
                        s16 = plsc.load_gather(clusv, [s16])
                    pos = (s16 * 9 + selw[sl]) * co + joff
                    d16 = dstw[sl]
                    inb = (d16 >= lo) & (d16 < lo + Nc)
                    m = inb.astype(I32)
                    pp = cnt + plsc.cumsum(m) - m
                    plsc.store_scatter(idxl, [pp], pos >> 7, mask=inb)
                    if packed:
                        plsc.store_scatter(ofl, [pp], pos & 127, mask=inb)
                    plsc.store_scatter(dl, [pp], (d16 - lo) * co_t, mask=inb)
                    plsc.store_scatter(itl, [pp], itpw[sl], mask=inb)
                    return cnt + jnp.sum(m)
                cnt = lax.fori_loop(0, ww // 16, _sc, 0)
                for u in range(LB // 16):
                    pv = cnt + u * 16 + IOTA
                    plsc.store_scatter(idxl, [pv], IOTA + u * 16)
                    if packed:
                        plsc.store_scatter(ofl, [pv], jnp.zeros((16,), I32))
                    plsc.store_scatter(dl, [pv], jnp.full((16,), Nc * co_t, I32))
                    plsc.store_scatter(itl, [pv], jnp.zeros((16,), F32))
                proc_blocks((cnt + LB - 1) // LB)

        for p in range(P):
            combo = p * 32 + wid
            j = combo % nsl
            h = (combo // nsl) % chunks
            g = combo // combos
            lo = h * Nc

            def _z(t, _):
                for u in range(8):
                    acc[pl.ds((t * 8 + u) * 16, 16)] = jnp.zeros((16,), F32)
                return 0
            lax.fori_loop(0, 512, _z, 0)

            gbase = g * Eg

            def _win(wi, _):
                scan_window(gbase + wi * W, W, j, lo)
                return 0
            lax.fori_loop(0, nwin, _win, 0)
            if rem:
                scan_window(gbase + nwin * W, rem, j, lo)

            row = (g * chunks + h) * nsl + j
            pltpu.sync_copy(acc.at[pl.ds(0, Nc * co_t)], out_r.at[row])

    scratch = []
    if has_clus:
        scratch.append(pltpu.VMEM((clus.shape[0],), I32))
    scratch += [
        pltpu.VMEM((W,), I32),
        pltpu.VMEM((W,), I32),
        pltpu.VMEM((W,), I32),
        pltpu.VMEM((W,), F32),
        pltpu.VMEM((LISTN,), I32),
        pltpu.VMEM((LISTN,), I32),
        pltpu.VMEM((LISTN,), F32),
        pltpu.VMEM((LISTN,), I32),
        pltpu.VMEM((LB, 128), F32),
        pltpu.VMEM((65536 + co_t,), F32),
        pltpu.SemaphoreType.DMA,
    ]
    mesh = plsc.VectorSubcoreMesh(core_axis_name="c", subcore_axis_name="s")
    fn = pl.kernel(
        body,
        out_type=jax.ShapeDtypeStruct((32 * P, Nc * co_t), F32),
        mesh=mesh,
        scratch_types=scratch,
        compiler_params=pltpu.CompilerParams(needs_layout_passes=False),
        name=f"sc_conv_N{N}_co{co}",
    )
    args = (tbl, src, sel, dst, itp) + ((clus,) if has_clus else ())
    return fn(*args)


def _assemble(p, N, co):
    co_t = min(co, 128)
    nsl = co // co_t
    Nc = min(N, 65536 // co_t)
    chunks = N // Nc
    G = p.shape[0] // (chunks * nsl)
    return (p.reshape(G, chunks, nsl, Nc, co_t)
            .transpose(0, 1, 3, 2, 4)
            .reshape(G, N, co))


def _sc_gather_i32(table, idx):
    E = idx.shape[0]
    per = E // 32
    nw = per // W

    def body(tbl_r, idx_r, out_r, idxw, valw, sem):
        cid = lax.axis_index("c")
        sid = lax.axis_index("s")
        base = (sid * 2 + cid) * per

        def _w(k, _):
            b = base + k * W
            pltpu.sync_copy(idx_r.at[pl.ds(b, W)], idxw)
            pltpu.async_copy(tbl_r.at[idxw], valw, sem).wait()
            pltpu.sync_copy(valw, out_r.at[pl.ds(b, W)])
            return 0
        lax.fori_loop(0, nw, _w, 0)

    mesh = plsc.VectorSubcoreMesh(core_axis_name="c", subcore_axis_name="s")
    fn = pl.kernel(
        body,
        out_type=jax.ShapeDtypeStruct((E,), I32),
        mesh=mesh,
        scratch_types=[
            pltpu.VMEM((W,), I32),
            pltpu.VMEM((W,), I32),
            pltpu.SemaphoreType.DMA,
        ],
        compiler_params=pltpu.CompilerParams(needs_layout_passes=False),
        name="sc_gather_i32",
    )
    return fn(table, idx)


def _mm_plain(x, w, bm):
    M, Kd = x.shape
    C = w.shape[1]

    def body(x_r, w_r, o_r):
        o_r[...] = jnp.dot(x_r[...], w_r[...], preferred_element_type=F32)

    return pl.pallas_call(
        body,
        grid=(M // bm,),
        in_specs=[pl.BlockSpec((bm, Kd), lambda i: (i, 0)),
                  pl.BlockSpec((Kd, C), lambda i: (0, 0))],
        out_specs=pl.BlockSpec((bm, C), lambda i: (i, 0)),
        out_shape=jax.ShapeDtypeStruct((M, C), F32),
    )(x, w)


def _mm_sum(p, b, w, bm):
    G, M, Kd = p.shape
    C = w.shape[1]
    b2 = b.reshape(1, Kd)

    def body(p_r, b_r, w_r, o_r):
        x = p_r[0]
        for g in range(1, G):
            x = x + p_r[g]
        x = jnp.maximum(x + b_r[0], 0.0)
        o_r[...] = jnp.dot(x, w_r[...], preferred_element_type=F32)

    return pl.pallas_call(
        body,
        grid=(M // bm,),
        in_specs=[pl.BlockSpec((G, bm, Kd), lambda i: (0, i, 0)),
                  pl.BlockSpec((1, Kd), lambda i: (0, 0)),
                  pl.BlockSpec((Kd, C), lambda i: (0, 0))],
        out_specs=pl.BlockSpec((bm, C), lambda i: (i, 0)),
        out_shape=jax.ShapeDtypeStruct((M, C), F32),
    )(p, b2, w)


def _final_sum(p, b, bm):
    G, M, C = p.shape
    b2 = b.reshape(1, C)

    def body(p_r, b_r, o_r):
        x = p_r[0]
        for g in range(1, G):
            x = x + p_r[g]
        o_r[...] = x + b_r[0]

    return pl.pallas_call(
        body,
        grid=(M // bm,),
        in_specs=[pl.BlockSpec((G, bm, C), lambda i: (0, i, 0)),
                  pl.BlockSpec((1, C), lambda i: (0, 0))],
        out_specs=pl.BlockSpec((bm, C), lambda i: (i, 0)),
        out_shape=jax.ShapeDtypeStruct((M, C), F32),
    )(p, b2)


def _wr(Wt, co=None):
    if co is not None and Wt.shape[2] != co:
        Wt = jnp.pad(Wt, ((0, 0), (0, 0), (0, co - Wt.shape[2])))
    ci = Wt.shape[1]
    return Wt.transpose(1, 0, 2).reshape(ci, 9 * Wt.shape[2])


def kernel(x, edge_index_3, selections_3, interps_3, edge_index_2, selections_2, interps_2, edge_index_1, selections_1, interps_1, edge_index_0, selections_0, interps_0, clusters_2, clusters_1, clusters_0, W11, b11, W12, b12, W13, b13, W14, b14, W15, b15, W16, b16, W17, b17, W18, b18, W19, b19):
    s3, d3 = edge_index_3[0], edge_index_3[1]
    s2, d2 = edge_index_2[0], edge_index_2[1]
    s1, d1 = edge_index_1[0], edge_index_1[1]
    s0, d0 = edge_index_0[0], edge_index_0[1]

    cs0 = _sc_gather_i32(clusters_0, s0)

    def tv(y):
        return y.reshape(-1, 128)

    y = _mm_plain(x, _wr(W11), 1024)
    p = _sc_conv(tv(y), s3, selections_3, d3, interps_3, None, 1024, 256)
    p = _assemble(p, 1024, 256)
    y = _mm_sum(p, b11, _wr(W12), 1024)
    p = _sc_conv(tv(y), s2, selections_2, d2, interps_2, clusters_2, 4096, 256)
    p = _assemble(p, 4096, 256)
    y = _mm_sum(p, b12, _wr(W13), 1024)
    p = _sc_conv(tv(y), s2, selections_2, d2, interps_2, None, 4096, 256)
    p = _assemble(p, 4096, 256)
    y = _mm_sum(p, b13, _wr(W14), 1024)
    p = _sc_conv(tv(y), s2, selections_2, d2, interps_2, None, 4096, 256)
    p = _assemble(p, 4096, 256)
    y = _mm_sum(p, b14, _wr(W15), 1024)
    p = _sc_conv(tv(y), s2, selections_2, d2, interps_2, None, 4096, 128)
    p = _assemble(p, 4096, 128)
    y = _mm_sum(p, b15, _wr(W16), 1024)
    p = _sc_conv(tv(y), s1, selections_1, d1, interps_1, clusters_1, 16384, 128)
    p = _assemble(p, 16384, 128)
    y = _mm_sum(p, b16, _wr(W17), 2048)
    p = _sc_conv(tv(y), s1, selections_1, d1, interps_1, None, 16384, 64)
    p = _assemble(p, 16384, 64)
    y = _mm_sum(p, b17, _wr(W18), 2048)
    p = _sc_conv(tv(y), cs0, selections_0, d0, interps_0, None, 65536, 64)
    p = _assemble(p, 65536, 64)
    y = _mm_sum(p, b18, _wr(W19, 16), 4096)
    p = _sc_conv(tv(y), s0, selections_0, d0, interps_0, None, 65536, 16)
    p = _assemble(p, 65536, 16)
    out = _final_sum(p, jnp.pad(b19, (0, 13)), 4096)
    return out[:, :3]
